# Initial kernel scaffold; baseline (speedup 1.0000x reference)
import jax
import jax.numpy as jnp
from jax.experimental import pallas as pl


def kernel(inputs, edge_index):
    raise NotImplementedError("write your pallas kernel here")


# speedup vs baseline: 1.4033x; 1.4033x over previous
import functools

import jax
import jax.numpy as jnp
from jax import lax
from jax.experimental import pallas as pl
from jax.experimental.pallas import tpu as pltpu
from jax.experimental.pallas import tpu_sc as plsc

N = 10000
E = 160000
D = 256
L = 16
NW = 32
NP = 320
CH = 4000
NCH = E // CH
NG = CH // L
B = 32
RB = 16
NEG = float("-inf")


def _gcn_body(x_hbm, src_hbm, dst_hbm, out_hbm,
              acc, srcb, dstb, gsrc, gdst, rowb, inb, outb,
              touched, sem):
  wid = lax.axis_index("s") * 2 + lax.axis_index("c")
  n0 = wid * NP

  neg = jnp.full((L,), NEG, jnp.float32)

  def init_acc(i, _):
    for j in range(D // L):
      acc[i, pl.ds(j * L, L)] = neg
    return 0
  lax.fori_loop(0, NP + 1, init_acc, 0)

  def init_touched(i, _):
    touched[i] = 0
    return 0
  lax.fori_loop(0, NP + 1, init_touched, 0)

  zero16 = jnp.zeros((L,), jnp.int32)
  trash16 = jnp.full((L,), NP, jnp.int32)

  def chunk_body(c, _):
    pltpu.sync_copy(src_hbm.at[pl.ds(c * CH, CH)], srcb)
    pltpu.sync_copy(dst_hbm.at[pl.ds(c * CH, CH)], dstb)

    def filt(g, cnt):
      d = dstb[pl.ds(g * L, L)]
      s = srcb[pl.ds(g * L, L)]
      dl = d - n0
      m = (dl >= 0) & (dl < NP)
      csum = plsc.cumsum(m.astype(jnp.int32))
      pos = cnt + csum - 1
      plsc.store_scatter(gsrc, [pos], s, mask=m)
      plsc.store_scatter(gdst, [pos], dl, mask=m)
      return cnt + csum[L - 1]

    cnt = lax.fori_loop(0, NG, filt, 0)

    gsrc[pl.ds(cnt, L)] = zero16
    gsrc[pl.ds(cnt + L, L)] = zero16
    gdst[pl.ds(cnt, L)] = trash16
    gdst[pl.ds(cnt + L, L)] = trash16

    nb = (cnt + B - 1) // B

    def gather_body(b, _):
      pltpu.async_copy(x_hbm.at[gsrc.at[pl.ds(b * B, B)]], rowb, sem).wait()
      for h in range(B // L):
        dlv = gdst[pl.ds(b * B + h * L, L)]
        for i in range(L):
          dl = dlv[i]
          touched[dl] = 1
          for j in range(D // L):
            sl = pl.ds(j * L, L)
            r = h * L + i
            acc[dl, sl] = jnp.maximum(acc[dl, sl], rowb[r, sl])
      return 0

    lax.fori_loop(0, nb, gather_body, 0)
    return 0

  lax.fori_loop(0, NCH, chunk_body, 0)

  def write_body(b, _):
    @pl.when(n0 + b * RB < N)
    def _():
      pltpu.sync_copy(x_hbm.at[pl.ds(n0 + b * RB, RB)], inb)
      for r in range(RB):
        t = touched[b * RB + r]
        tv = lax.broadcast_in_dim(t, (L,), ()) > 0
        for j in range(D // L):
          iv = inb[r, pl.ds(j * L, L)]
          av = acc[b * RB + r, pl.ds(j * L, L)]
          outb[r, pl.ds(j * L, L)] = iv
          outb[r, pl.ds(D + j * L, L)] = jnp.where(tv, av, iv)
      pltpu.sync_copy(outb, out_hbm.at[pl.ds(n0 + b * RB, RB)])
    return 0

  lax.fori_loop(0, NP // RB, write_body, 0)


@functools.partial(jax.jit, donate_argnums=())
def _gcn(x, src, dst):
  mesh = plsc.VectorSubcoreMesh(core_axis_name="c", subcore_axis_name="s")
  run = pl.kernel(
      _gcn_body,
      compiler_params=pltpu.CompilerParams(needs_layout_passes=False),
      out_type=jax.ShapeDtypeStruct((N, 2 * D), jnp.float32),
      mesh=mesh,
      scratch_types=[
          pltpu.VMEM((NP + 1, D), jnp.float32),
          pltpu.VMEM((CH,), jnp.int32),
          pltpu.VMEM((CH,), jnp.int32),
          pltpu.VMEM((CH + B,), jnp.int32),
          pltpu.VMEM((CH + B,), jnp.int32),
          pltpu.VMEM((B, D), jnp.float32),
          pltpu.VMEM((RB, D), jnp.float32),
          pltpu.VMEM((RB, 2 * D), jnp.float32),
          pltpu.SMEM((NP + 1,), jnp.int32),
          pltpu.SemaphoreType.DMA,
      ],
  )
  return run(x, src, dst)


def kernel(inputs, edge_index):
  return _gcn(inputs, edge_index[0], edge_index[1])


# speedup vs baseline: 1.4269x; 1.0168x over previous
import functools

import jax
import jax.numpy as jnp
from jax import lax
from jax.experimental import pallas as pl
from jax.experimental.pallas import tpu as pltpu
from jax.experimental.pallas import tpu_sc as plsc

N = 10000
E = 160000
D = 256
L = 16
NW = 32
NP = 320
CH = 4000
NCH = E // CH
NG = CH // L
B = 32
RB = 8
NEG = float("-inf")


def _gcn_body(x_hbm, src_hbm, dst_hbm, out_hbm,
              acc, srcb0, dstb0, srcb1, dstb1, gsrc, gdst,
              rowb0, rowb1, inb, outb,
              touched, esem0, esem1, gsem0, gsem1):
  wid = lax.axis_index("s") * 2 + lax.axis_index("c")
  n0 = wid * NP

  neg = jnp.full((L,), NEG, jnp.float32)

  def init_acc(i, _):
    for j in range(D // L):
      acc[i, pl.ds(j * L, L)] = neg
    return 0
  lax.fori_loop(0, NP + 1, init_acc, 0)

  def init_touched(i, _):
    touched[i] = 0
    return 0
  lax.fori_loop(0, NP + 1, init_touched, 0)

  zero16 = jnp.zeros((L,), jnp.int32)
  trash16 = jnp.full((L,), NP, jnp.int32)

  def fire_edges(c, sb, db, sem):
    pltpu.async_copy(src_hbm.at[pl.ds(c * CH, CH)], sb, sem)
    pltpu.async_copy(dst_hbm.at[pl.ds(c * CH, CH)], db, sem)

  def wait_edges(c, sb, db, sem):
    pltpu.make_async_copy(src_hbm.at[pl.ds(c * CH, CH)], sb, sem).wait()
    pltpu.make_async_copy(dst_hbm.at[pl.ds(c * CH, CH)], db, sem).wait()

  def fire_gather(b, rb, sem):
    pltpu.async_copy(x_hbm.at[gsrc.at[pl.ds(b * B, B)]], rb, sem)

  def wait_gather(b, rb, sem):
    pltpu.make_async_copy(x_hbm.at[gsrc.at[pl.ds(b * B, B)]], rb, sem).wait()

  def process_chunk(c, sb, db):
    def filt2(g2, cnt):
      g = g2 * 2
      d0 = db[pl.ds(g * L, L)]
      s0 = sb[pl.ds(g * L, L)]
      d1 = db[pl.ds((g + 1) * L, L)]
      s1 = sb[pl.ds((g + 1) * L, L)]
      dl0 = d0 - n0
      dl1 = d1 - n0
      m0 = (dl0 >= 0) & (dl0 < NP)
      m1 = (dl1 >= 0) & (dl1 < NP)
      csum0 = plsc.cumsum(jnp.where(m0, 1, 0))
      csum1 = plsc.cumsum(jnp.where(m1, 1, 0))
      pc0 = plsc.all_reduce_population_count(m0)[0]
      pc1 = plsc.all_reduce_population_count(m1)[0]
      pos0 = cnt + csum0 - 1
      plsc.store_scatter(gsrc, [pos0], s0, mask=m0)
      plsc.store_scatter(gdst, [pos0], dl0, mask=m0)
      cnt1 = cnt + pc0
      pos1 = cnt1 + csum1 - 1
      plsc.store_scatter(gsrc, [pos1], s1, mask=m1)
      plsc.store_scatter(gdst, [pos1], dl1, mask=m1)
      return cnt1 + pc1

    cnt = lax.fori_loop(0, NG // 2, filt2, 0)

    gsrc[pl.ds(cnt, L)] = zero16
    gsrc[pl.ds(cnt + L, L)] = zero16
    gdst[pl.ds(cnt, L)] = trash16
    gdst[pl.ds(cnt + L, L)] = trash16

    nb = (cnt + B - 1) // B

    def apply(b, rb):
      for h in range(B // L):
        dlv = gdst[pl.ds(b * B + h * L, L)]
        for i in range(L):
          dl = dlv[i]
          touched[dl] = 1
          for j in range(D // L):
            sl = pl.ds(j * L, L)
            r = h * L + i
            acc[dl, sl] = jnp.maximum(acc[dl, sl], rb[r, sl])

    @pl.when(nb > 0)
    def _():
      fire_gather(0, rowb0, gsem0)

      def gpair(bb, _):
        b0 = bb * 2
        wait_gather(b0, rowb0, gsem0)

        @pl.when(b0 + 1 < nb)
        def _():
          fire_gather(b0 + 1, rowb1, gsem1)
        apply(b0, rowb0)

        @pl.when(b0 + 1 < nb)
        def _():
          wait_gather(b0 + 1, rowb1, gsem1)

          @pl.when(b0 + 2 < nb)
          def _():
            fire_gather(b0 + 2, rowb0, gsem0)
          apply(b0 + 1, rowb1)
        return 0

      lax.fori_loop(0, (nb + 1) // 2, gpair, 0)

  fire_edges(0, srcb0, dstb0, esem0)

  def cpair(cc, _):
    c0 = cc * 2
    wait_edges(c0, srcb0, dstb0, esem0)
    fire_edges(c0 + 1, srcb1, dstb1, esem1)
    process_chunk(c0, srcb0, dstb0)
    wait_edges(c0 + 1, srcb1, dstb1, esem1)

    @pl.when(c0 + 2 < NCH)
    def _():
      fire_edges(c0 + 2, srcb0, dstb0, esem0)
    process_chunk(c0 + 1, srcb1, dstb1)
    return 0

  lax.fori_loop(0, NCH // 2, cpair, 0)

  def write_body(b, _):
    @pl.when(n0 + b * RB < N)
    def _():
      pltpu.sync_copy(x_hbm.at[pl.ds(n0 + b * RB, RB)], inb)
      for r in range(RB):
        t = touched[b * RB + r]
        tv = lax.broadcast_in_dim(t, (L,), ()) > 0
        for j in range(D // L):
          iv = inb[r, pl.ds(j * L, L)]
          av = acc[b * RB + r, pl.ds(j * L, L)]
          outb[r, pl.ds(j * L, L)] = iv
          outb[r, pl.ds(D + j * L, L)] = jnp.where(tv, av, iv)
      pltpu.sync_copy(outb, out_hbm.at[pl.ds(n0 + b * RB, RB)])
    return 0

  lax.fori_loop(0, NP // RB, write_body, 0)


@functools.partial(jax.jit, donate_argnums=())
def _gcn(x, src, dst):
  mesh = plsc.VectorSubcoreMesh(core_axis_name="c", subcore_axis_name="s")
  run = pl.kernel(
      _gcn_body,
      compiler_params=pltpu.CompilerParams(needs_layout_passes=False),
      out_type=jax.ShapeDtypeStruct((N, 2 * D), jnp.float32),
      mesh=mesh,
      scratch_types=[
          pltpu.VMEM((NP + 1, D), jnp.float32),
          pltpu.VMEM((CH,), jnp.int32),
          pltpu.VMEM((CH,), jnp.int32),
          pltpu.VMEM((CH,), jnp.int32),
          pltpu.VMEM((CH,), jnp.int32),
          pltpu.VMEM((CH + B,), jnp.int32),
          pltpu.VMEM((CH + B,), jnp.int32),
          pltpu.VMEM((B, D), jnp.float32),
          pltpu.VMEM((B, D), jnp.float32),
          pltpu.VMEM((RB, D), jnp.float32),
          pltpu.VMEM((RB, 2 * D), jnp.float32),
          pltpu.SMEM((NP + 1,), jnp.int32),
          pltpu.SemaphoreType.DMA,
          pltpu.SemaphoreType.DMA,
          pltpu.SemaphoreType.DMA,
          pltpu.SemaphoreType.DMA,
      ],
  )
  return run(x, src, dst)


def kernel(inputs, edge_index):
  return _gcn(inputs, edge_index[0], edge_index[1])


# speedup vs baseline: 1.6491x; 1.1557x over previous
import functools

import jax
import jax.numpy as jnp
from jax import lax
from jax.experimental import pallas as pl
from jax.experimental.pallas import tpu as pltpu
from jax.experimental.pallas import tpu_sc as plsc

N = 10000
E = 160000
D = 256
L = 16
NW = 32
NP = 320
CH = 4000
NCH = E // CH
NG = CH // L
B = 32
RB = 8
NEG = float("-inf")


def _gcn_body(x_hbm, src_hbm, dst_hbm, out_hbm,
              acc, srcb0, dstb0, srcb1, dstb1, gsrc, gdst,
              rowb0, rowb1, inb, outb,
              touched, esem0, esem1, gsem0, gsem1):
  wid = lax.axis_index("s") * 2 + lax.axis_index("c")
  n0 = wid * NP

  neg = jnp.full((L,), NEG, jnp.float32)

  def init_acc(i, _):
    for j in range(D // L):
      acc[i, pl.ds(j * L, L)] = neg
    return 0
  lax.fori_loop(0, NP + 1, init_acc, 0)

  def init_touched(i, _):
    touched[i] = 0
    return 0
  lax.fori_loop(0, NP + 1, init_touched, 0)

  zero16 = jnp.zeros((L,), jnp.int32)
  trash16 = jnp.full((L,), NP, jnp.int32)

  def fire_edges(c, sb, db, sem):
    pltpu.async_copy(src_hbm.at[pl.ds(c * CH, CH)], sb, sem)
    pltpu.async_copy(dst_hbm.at[pl.ds(c * CH, CH)], db, sem)

  def wait_edges(c, sb, db, sem):
    pltpu.make_async_copy(src_hbm.at[pl.ds(c * CH, CH)], sb, sem).wait()
    pltpu.make_async_copy(dst_hbm.at[pl.ds(c * CH, CH)], db, sem).wait()

  def fire_gather(b, rb, sem):
    pltpu.async_copy(x_hbm.at[gsrc.at[pl.ds(b * B, B)]], rb, sem)

  def wait_gather(b, rb, sem):
    pltpu.make_async_copy(x_hbm.at[gsrc.at[pl.ds(b * B, B)]], rb, sem).wait()

  def process_chunk(c, sb, db):
    npv = jnp.full((L,), NP, jnp.uint32)

    def filt2(g2, cntv):
      g = g2 * 2
      d0 = db[pl.ds(g * L, L)]
      s0 = sb[pl.ds(g * L, L)]
      d1 = db[pl.ds((g + 1) * L, L)]
      s1 = sb[pl.ds((g + 1) * L, L)]
      dl0 = d0 - n0
      dl1 = d1 - n0
      m0 = plsc.bitcast(dl0, jnp.uint32) < npv
      m1 = plsc.bitcast(dl1, jnp.uint32) < npv
      csum0 = plsc.cumsum(jnp.where(m0, 1, 0))
      csum1 = plsc.cumsum(jnp.where(m1, 1, 0))
      pc0v = plsc.all_reduce_population_count(m0)
      pc1v = plsc.all_reduce_population_count(m1)
      pos0 = cntv + csum0 - 1
      plsc.store_scatter(gsrc, [pos0], s0, mask=m0)
      plsc.store_scatter(gdst, [pos0], dl0, mask=m0)
      cnt1v = cntv + pc0v
      pos1 = cnt1v + csum1 - 1
      plsc.store_scatter(gsrc, [pos1], s1, mask=m1)
      plsc.store_scatter(gdst, [pos1], dl1, mask=m1)
      return cnt1v + pc1v

    cntv = lax.fori_loop(0, NG // 2, filt2, jnp.zeros((L,), jnp.int32))
    cnt = cntv[0]

    gsrc[pl.ds(cnt, L)] = zero16
    gsrc[pl.ds(cnt + L, L)] = zero16
    gdst[pl.ds(cnt, L)] = trash16
    gdst[pl.ds(cnt + L, L)] = trash16

    nb = (cnt + B - 1) // B

    def apply(b, rb):
      for h in range(B // L):
        dlv = gdst[pl.ds(b * B + h * L, L)]
        for i in range(L):
          dl = dlv[i]
          touched[dl] = 1
          r = h * L + i
          avs = [acc[dl, pl.ds(j * L, L)] for j in range(D // L)]
          rvs = [rb[r, pl.ds(j * L, L)] for j in range(D // L)]
          for j in range(D // L):
            acc[dl, pl.ds(j * L, L)] = jnp.maximum(avs[j], rvs[j])

    @pl.when(nb > 0)
    def _():
      fire_gather(0, rowb0, gsem0)

      def gpair(bb, _):
        b0 = bb * 2
        wait_gather(b0, rowb0, gsem0)

        @pl.when(b0 + 1 < nb)
        def _():
          fire_gather(b0 + 1, rowb1, gsem1)
        apply(b0, rowb0)

        @pl.when(b0 + 1 < nb)
        def _():
          wait_gather(b0 + 1, rowb1, gsem1)

          @pl.when(b0 + 2 < nb)
          def _():
            fire_gather(b0 + 2, rowb0, gsem0)
          apply(b0 + 1, rowb1)
        return 0

      lax.fori_loop(0, (nb + 1) // 2, gpair, 0)

  fire_edges(0, srcb0, dstb0, esem0)

  def cpair(cc, _):
    c0 = cc * 2
    wait_edges(c0, srcb0, dstb0, esem0)
    fire_edges(c0 + 1, srcb1, dstb1, esem1)
    process_chunk(c0, srcb0, dstb0)
    wait_edges(c0 + 1, srcb1, dstb1, esem1)

    @pl.when(c0 + 2 < NCH)
    def _():
      fire_edges(c0 + 2, srcb0, dstb0, esem0)
    process_chunk(c0 + 1, srcb1, dstb1)
    return 0

  lax.fori_loop(0, NCH // 2, cpair, 0)

  def write_body(b, _):
    @pl.when(n0 + b * RB < N)
    def _():
      pltpu.sync_copy(x_hbm.at[pl.ds(n0 + b * RB, RB)], inb)
      for r in range(RB):
        t = touched[b * RB + r]
        tv = lax.broadcast_in_dim(t, (L,), ()) > 0
        for j in range(D // L):
          iv = inb[r, pl.ds(j * L, L)]
          av = acc[b * RB + r, pl.ds(j * L, L)]
          outb[r, pl.ds(j * L, L)] = iv
          outb[r, pl.ds(D + j * L, L)] = jnp.where(tv, av, iv)
      pltpu.sync_copy(outb, out_hbm.at[pl.ds(n0 + b * RB, RB)])
    return 0

  lax.fori_loop(0, NP // RB, write_body, 0)


@functools.partial(jax.jit, donate_argnums=())
def _gcn(x, src, dst):
  mesh = plsc.VectorSubcoreMesh(core_axis_name="c", subcore_axis_name="s")
  run = pl.kernel(
      _gcn_body,
      compiler_params=pltpu.CompilerParams(needs_layout_passes=False),
      out_type=jax.ShapeDtypeStruct((N, 2 * D), jnp.float32),
      mesh=mesh,
      scratch_types=[
          pltpu.VMEM((NP + 1, D), jnp.float32),
          pltpu.VMEM((CH,), jnp.int32),
          pltpu.VMEM((CH,), jnp.int32),
          pltpu.VMEM((CH,), jnp.int32),
          pltpu.VMEM((CH,), jnp.int32),
          pltpu.VMEM((CH + B,), jnp.int32),
          pltpu.VMEM((CH + B,), jnp.int32),
          pltpu.VMEM((B, D), jnp.float32),
          pltpu.VMEM((B, D), jnp.float32),
          pltpu.VMEM((RB, D), jnp.float32),
          pltpu.VMEM((RB, 2 * D), jnp.float32),
          pltpu.SMEM((NP + 1,), jnp.int32),
          pltpu.SemaphoreType.DMA,
          pltpu.SemaphoreType.DMA,
          pltpu.SemaphoreType.DMA,
          pltpu.SemaphoreType.DMA,
      ],
  )
  return run(x, src, dst)


def kernel(inputs, edge_index):
  return _gcn(inputs, edge_index[0], edge_index[1])


# speedup vs baseline: 1.6499x; 1.0005x over previous
import functools

import jax
import jax.numpy as jnp
from jax import lax
from jax.experimental import pallas as pl
from jax.experimental.pallas import tpu as pltpu
from jax.experimental.pallas import tpu_sc as plsc

N = 10000
E = 160000
D = 256
L = 16
NW = 32
NP = 320
CH = 4000
NCH = E // CH
NG = CH // L
B = 32
RB = 8
NEG = float("-inf")


def _gcn_body(x_hbm, src_hbm, dst_hbm, out_hbm,
              acc, srcb0, dstb0, srcb1, dstb1, gsrc, gdst,
              rowb0, rowb1, inb, outb,
              touched, esem0, esem1, gsem0, gsem1):
  wid = lax.axis_index("s") * 2 + lax.axis_index("c")
  n0 = wid * NP

  neg = jnp.full((L,), NEG, jnp.float32)

  def init_acc(i, _):
    for j in range(D // L):
      acc[i, pl.ds(j * L, L)] = neg
    return 0
  lax.fori_loop(0, NP + 1, init_acc, 0)

  def init_touched(i, _):
    touched[i] = 0
    return 0
  lax.fori_loop(0, NP + 1, init_touched, 0)

  zero16 = jnp.zeros((L,), jnp.int32)
  trash16 = jnp.full((L,), NP, jnp.int32)

  def fire_edges(c, sb, db, sem):
    pltpu.async_copy(src_hbm.at[pl.ds(c * CH, CH)], sb, sem)
    pltpu.async_copy(dst_hbm.at[pl.ds(c * CH, CH)], db, sem)

  def wait_edges(c, sb, db, sem):
    pltpu.make_async_copy(src_hbm.at[pl.ds(c * CH, CH)], sb, sem).wait()
    pltpu.make_async_copy(dst_hbm.at[pl.ds(c * CH, CH)], db, sem).wait()

  def fire_gather(b, rb, sem):
    pltpu.async_copy(x_hbm.at[gsrc.at[pl.ds(b * B, B)]], rb, sem)

  def wait_gather(b, rb, sem):
    pltpu.make_async_copy(x_hbm.at[gsrc.at[pl.ds(b * B, B)]], rb, sem).wait()

  def process_chunk(c, sb, db):
    npv = jnp.full((L,), NP, jnp.uint32)

    def filt2(g2, cntv):
      g = g2 * 2
      d0 = db[pl.ds(g * L, L)]
      s0 = sb[pl.ds(g * L, L)]
      d1 = db[pl.ds((g + 1) * L, L)]
      s1 = sb[pl.ds((g + 1) * L, L)]
      dl0 = d0 - n0
      dl1 = d1 - n0
      m0 = plsc.bitcast(dl0, jnp.uint32) < npv
      m1 = plsc.bitcast(dl1, jnp.uint32) < npv
      csum0 = plsc.cumsum(jnp.where(m0, 1, 0))
      csum1 = plsc.cumsum(jnp.where(m1, 1, 0))
      pc0v = plsc.all_reduce_population_count(m0)
      pc1v = plsc.all_reduce_population_count(m1)
      pos0 = cntv + csum0 - 1
      plsc.store_scatter(gsrc, [pos0], s0, mask=m0)
      plsc.store_scatter(gdst, [pos0], dl0, mask=m0)
      cnt1v = cntv + pc0v
      pos1 = cnt1v + csum1 - 1
      plsc.store_scatter(gsrc, [pos1], s1, mask=m1)
      plsc.store_scatter(gdst, [pos1], dl1, mask=m1)
      return cnt1v + pc1v

    with jax.named_scope("filt"):
      cntv = lax.fori_loop(0, NG // 2, filt2, jnp.zeros((L,), jnp.int32))
    cnt = cntv[0]

    gsrc[pl.ds(cnt, L)] = zero16
    gsrc[pl.ds(cnt + L, L)] = zero16
    gdst[pl.ds(cnt, L)] = trash16
    gdst[pl.ds(cnt + L, L)] = trash16

    nb = (cnt + B - 1) // B

    def apply(b, rb):
      for h in range(B // L):
        dlv = gdst[pl.ds(b * B + h * L, L)]
        for i in range(L):
          dl = dlv[i]
          touched[dl] = 1
          r = h * L + i
          avs = [acc[dl, pl.ds(j * L, L)] for j in range(D // L)]
          rvs = [rb[r, pl.ds(j * L, L)] for j in range(D // L)]
          for j in range(D // L):
            acc[dl, pl.ds(j * L, L)] = jnp.maximum(avs[j], rvs[j])

    @pl.when(nb > 0)
    def _():
     with jax.named_scope("gap"):
      fire_gather(0, rowb0, gsem0)

      def gpair(bb, _):
        b0 = bb * 2
        wait_gather(b0, rowb0, gsem0)

        @pl.when(b0 + 1 < nb)
        def _():
          fire_gather(b0 + 1, rowb1, gsem1)
        apply(b0, rowb0)

        @pl.when(b0 + 1 < nb)
        def _():
          wait_gather(b0 + 1, rowb1, gsem1)

          @pl.when(b0 + 2 < nb)
          def _():
            fire_gather(b0 + 2, rowb0, gsem0)
          apply(b0 + 1, rowb1)
        return 0

      lax.fori_loop(0, (nb + 1) // 2, gpair, 0)

  fire_edges(0, srcb0, dstb0, esem0)

  def cpair(cc, _):
    c0 = cc * 2
    wait_edges(c0, srcb0, dstb0, esem0)
    fire_edges(c0 + 1, srcb1, dstb1, esem1)
    process_chunk(c0, srcb0, dstb0)
    wait_edges(c0 + 1, srcb1, dstb1, esem1)

    @pl.when(c0 + 2 < NCH)
    def _():
      fire_edges(c0 + 2, srcb0, dstb0, esem0)
    process_chunk(c0 + 1, srcb1, dstb1)
    return 0

  lax.fori_loop(0, NCH // 2, cpair, 0)

  def write_body(b, _):
   with jax.named_scope("wout"):
    @pl.when(n0 + b * RB < N)
    def _():
      pltpu.sync_copy(x_hbm.at[pl.ds(n0 + b * RB, RB)], inb)
      for r in range(RB):
        t = touched[b * RB + r]
        tv = lax.broadcast_in_dim(t, (L,), ()) > 0
        for j in range(D // L):
          iv = inb[r, pl.ds(j * L, L)]
          av = acc[b * RB + r, pl.ds(j * L, L)]
          outb[r, pl.ds(j * L, L)] = iv
          outb[r, pl.ds(D + j * L, L)] = jnp.where(tv, av, iv)
      pltpu.sync_copy(outb, out_hbm.at[pl.ds(n0 + b * RB, RB)])
    return 0

  lax.fori_loop(0, NP // RB, write_body, 0)


@functools.partial(jax.jit, donate_argnums=())
def _gcn(x, src, dst):
  mesh = plsc.VectorSubcoreMesh(core_axis_name="c", subcore_axis_name="s")
  run = pl.kernel(
      _gcn_body,
      compiler_params=pltpu.CompilerParams(needs_layout_passes=False),
      out_type=jax.ShapeDtypeStruct((N, 2 * D), jnp.float32),
      mesh=mesh,
      scratch_types=[
          pltpu.VMEM((NP + 1, D), jnp.float32),
          pltpu.VMEM((CH,), jnp.int32),
          pltpu.VMEM((CH,), jnp.int32),
          pltpu.VMEM((CH,), jnp.int32),
          pltpu.VMEM((CH,), jnp.int32),
          pltpu.VMEM((CH + B,), jnp.int32),
          pltpu.VMEM((CH + B,), jnp.int32),
          pltpu.VMEM((B, D), jnp.float32),
          pltpu.VMEM((B, D), jnp.float32),
          pltpu.VMEM((RB, D), jnp.float32),
          pltpu.VMEM((RB, 2 * D), jnp.float32),
          pltpu.SMEM((NP + 1,), jnp.int32),
          pltpu.SemaphoreType.DMA,
          pltpu.SemaphoreType.DMA,
          pltpu.SemaphoreType.DMA,
          pltpu.SemaphoreType.DMA,
      ],
  )
  return run(x, src, dst)


def kernel(inputs, edge_index):
  return _gcn(inputs, edge_index[0], edge_index[1])
